# Initial kernel scaffold; baseline (speedup 1.0000x reference)
import jax
import jax.numpy as jnp
from jax.experimental import pallas as pl


def kernel(xyz, canno_xyz, radius):
    raise NotImplementedError("write your pallas kernel here")


# speedup vs baseline: 58.6252x; 58.6252x over previous
import functools

import jax
import jax.numpy as jnp
from jax.experimental import pallas as pl

_KEEP = 19.0
_HALF_TEMP = 0.05


def _loss_kernel(r2_ref, xi_ref, xt_ref, ci_ref, ct_ref, u_ref,
                 out_ref, *, bi, cj, n, b):
    step = pl.program_id(0)

    @pl.when(step == 0)
    def _():
        out_ref[...] = jnp.zeros_like(out_ref)

    r2 = r2_ref[0, 0]
    rows = step * bi + jax.lax.broadcasted_iota(jnp.int32, (bi, 1), 0)
    cols = jax.lax.broadcasted_iota(jnp.int32, (bi, n), 1)

    ci = ci_ref[...]
    d2c = jnp.zeros((bi, n), jnp.float32)
    for d in range(3):
        diff = ci[:, d:d + 1] - ct_ref[d:d + 1, :]
        d2c = d2c + diff * diff
    cnd = jnp.sqrt(d2c)
    sw = jnp.exp(d2c * (-_HALF_TEMP))

    u = u_ref[...]
    accv = jnp.zeros((bi, cj), jnp.float32)
    for bb in range(b):
        xi = xi_ref[bb]
        d2x = jnp.zeros((bi, n), jnp.float32)
        for d in range(3):
            diff = xi[:, d:d + 1] - xt_ref[bb][d:d + 1, :]
            d2x = d2x + diff * diff
        within = (d2x < r2) & (cols != rows)
        wf = within.astype(jnp.bfloat16)
        cd = jnp.sqrt(d2x)
        prior = jnp.zeros((bi, 1), jnp.float32)
        for c in range(n // cj):
            sl = slice(c * cj, (c + 1) * cj)
            incl = jax.lax.dot_general(
                wf[:, sl], u, (((1,), (0,)), ((), ())),
                preferred_element_type=jnp.float32)
            sel = within[:, sl] & ((prior + incl) <= _KEEP)
            t = jnp.abs(cd[:, sl] - cnd[:, sl]) * sw[:, sl] - 1e-10
            accv = accv + jnp.where(sel, t, 0.0)
            prior = prior + incl[:, cj - 1:cj]
    out_ref[...] += jnp.sum(accv, keepdims=True)


def kernel(xyz, canno_xyz, radius):
    b, n, _ = xyz.shape
    bi, cj = 128, 512
    xt = jnp.swapaxes(xyz, 1, 2)
    ct = canno_xyz.T
    r = jnp.asarray(radius, jnp.float32)
    r2 = jnp.reshape(r * r, (1, 1))
    u = jnp.triu(jnp.ones((cj, cj), jnp.bfloat16))
    out = pl.pallas_call(
        functools.partial(_loss_kernel, bi=bi, cj=cj, n=n, b=b),
        grid=(n // bi,),
        in_specs=[
            pl.BlockSpec((1, 1), lambda i: (0, 0)),
            pl.BlockSpec((b, bi, 3), lambda i: (0, i, 0)),
            pl.BlockSpec((b, 3, n), lambda i: (0, 0, 0)),
            pl.BlockSpec((bi, 3), lambda i: (i, 0)),
            pl.BlockSpec((3, n), lambda i: (0, 0)),
            pl.BlockSpec((cj, cj), lambda i: (0, 0)),
        ],
        out_specs=pl.BlockSpec((1, 1), lambda i: (0, 0)),
        out_shape=jax.ShapeDtypeStruct((1, 1), jnp.float32),
    )(r2, xyz, xt, canno_xyz, ct, u)
    denom = b * n * int(_KEEP)
    return (out[0, 0] / denom + 1e-10).astype(jnp.float32)


# speedup vs baseline: 120.0940x; 2.0485x over previous
import functools

import jax
import jax.numpy as jnp
from jax.experimental import pallas as pl

_KEEP = 19.0
_HALF_TEMP = 0.05


def _loss_kernel(r2_ref, xa_ref, xbt_ref, ca_ref, cbt_ref, u_ref,
                 out_ref, *, bi, cj, n, b):
    step = pl.program_id(0)

    @pl.when(step == 0)
    def _():
        out_ref[...] = jnp.zeros_like(out_ref)

    r2 = r2_ref[0, 0]
    rows = step * bi + jax.lax.broadcasted_iota(jnp.int32, (bi, 1), 0)
    cols = jax.lax.broadcasted_iota(jnp.int32, (bi, n), 1)
    noteye = jnp.where(cols == rows, 0.0, 1.0)

    zero = jnp.bfloat16(0.0)
    one = jnp.bfloat16(1.0)
    twenty = jnp.bfloat16(20.0)
    u = u_ref[...]
    dn = (((1,), (0,)), ((), ()))

    d2c = jax.lax.dot_general(ca_ref[...], cbt_ref[...], dn,
                              preferred_element_type=jnp.float32
                              ).astype(jnp.bfloat16)
    tiny = jnp.bfloat16(1e-8)
    dcp = jnp.maximum(d2c, tiny)
    cnd = dcp * jax.lax.rsqrt(dcp)
    sw = jnp.exp(d2c * jnp.bfloat16(-_HALF_TEMP))

    accv = jnp.zeros((bi, cj), jnp.bfloat16)
    scnt = jnp.zeros((bi, 1), jnp.bfloat16)
    for bb in range(b):
        xab = xa_ref[bb]
        prior = jnp.zeros((bi, 1), jnp.bfloat16)
        for c in range(n // cj):
            sl = slice(c * cj, (c + 1) * cj)
            d2x = jax.lax.dot_general(xab, xbt_ref[bb][:, sl], dn,
                                      preferred_element_type=jnp.float32)
            wc = jnp.where(d2x < r2, noteye[:, sl], 0.0).astype(jnp.bfloat16)
            dxp = jnp.maximum(d2x.astype(jnp.bfloat16), tiny)
            tall = jnp.abs(dxp * jax.lax.rsqrt(dxp) - cnd[:, sl]) * sw[:, sl]
            incl = jax.lax.dot_general(wc, u, dn,
                                       preferred_element_type=jnp.float32
                                       ).astype(jnp.bfloat16)
            ind = wc * jnp.clip(twenty - (prior + incl), zero, one)
            accv = accv + tall * ind
            prior = jnp.minimum(prior + incl[:, cj - 1:cj], twenty)
        scnt = scnt + jnp.minimum(prior, jnp.bfloat16(_KEEP))
    out_ref[...] += (jnp.sum(accv.astype(jnp.float32), keepdims=True)
                     - 1e-10 * jnp.sum(scnt.astype(jnp.float32), keepdims=True))


def kernel(xyz, canno_xyz, radius):
    b, n, _ = xyz.shape
    bi, cj = 256, 256
    sq = jnp.sum(xyz * xyz, axis=-1, keepdims=True)
    ones = jnp.ones_like(sq)
    xa = jnp.concatenate([xyz, sq, ones], axis=-1)
    xbt = jnp.swapaxes(
        jnp.concatenate([-2.0 * xyz, ones, sq], axis=-1), 1, 2)
    xah = xa.astype(jnp.bfloat16)
    xal = (xa - xah.astype(jnp.float32)).astype(jnp.bfloat16)
    xbh = xbt.astype(jnp.bfloat16)
    xbl = (xbt - xbh.astype(jnp.float32)).astype(jnp.bfloat16)
    xa3 = jnp.concatenate([xah, xah, xal], axis=-1)
    xb3 = jnp.concatenate([xbh, xbl, xbh], axis=1)
    csq = jnp.sum(canno_xyz * canno_xyz, axis=-1, keepdims=True)
    cones = jnp.ones_like(csq)
    ca = jnp.concatenate([canno_xyz, csq, cones],
                         axis=-1).astype(jnp.bfloat16)
    cbt = jnp.concatenate([-2.0 * canno_xyz, cones, csq],
                          axis=-1).T.astype(jnp.bfloat16)
    r = jnp.asarray(radius, jnp.float32)
    r2 = jnp.reshape(r * r, (1, 1))
    u = jnp.triu(jnp.ones((cj, cj), jnp.bfloat16))
    out = pl.pallas_call(
        functools.partial(_loss_kernel, bi=bi, cj=cj, n=n, b=b),
        grid=(n // bi,),
        in_specs=[
            pl.BlockSpec((1, 1), lambda i: (0, 0)),
            pl.BlockSpec((b, bi, 15), lambda i: (0, i, 0)),
            pl.BlockSpec((b, 15, n), lambda i: (0, 0, 0)),
            pl.BlockSpec((bi, 5), lambda i: (i, 0)),
            pl.BlockSpec((5, n), lambda i: (0, 0)),
            pl.BlockSpec((cj, cj), lambda i: (0, 0)),
        ],
        out_specs=pl.BlockSpec((1, 1), lambda i: (0, 0)),
        out_shape=jax.ShapeDtypeStruct((1, 1), jnp.float32),
    )(r2, xa3, xb3, ca, cbt, u)
    denom = b * n * int(_KEEP)
    return (out[0, 0] / denom + 1e-10).astype(jnp.float32)


# speedup vs baseline: 125.0118x; 1.0409x over previous
import functools

import jax
import jax.numpy as jnp
from jax.experimental import pallas as pl

_KEEP = 19.0
_HALF_TEMP = 0.05


def _loss_kernel(r2_ref, xa_ref, xbt_ref, ca_ref, cbt_ref, u_ref,
                 out_ref, *, bi, cj, n, b):
    step = pl.program_id(0)

    @pl.when(step == 0)
    def _():
        out_ref[...] = jnp.zeros_like(out_ref)

    r2 = r2_ref[0, 0]
    rows = step * bi + jax.lax.broadcasted_iota(jnp.int32, (bi, 1), 0)
    cols = jax.lax.broadcasted_iota(jnp.int32, (bi, n), 1)
    noteye = jnp.where(cols == rows, 0.0, 1.0)

    zero = jnp.bfloat16(0.0)
    one = jnp.bfloat16(1.0)
    twenty = jnp.bfloat16(20.0)
    u = u_ref[...]
    dn = (((1,), (0,)), ((), ()))

    d2c = jax.lax.dot_general(ca_ref[...], cbt_ref[...], dn,
                              preferred_element_type=jnp.float32
                              ).astype(jnp.bfloat16)
    tiny = jnp.bfloat16(1e-8)
    dcp = jnp.maximum(d2c, tiny)
    cnd = dcp * jax.lax.rsqrt(dcp)
    sw = jnp.exp(d2c * jnp.bfloat16(-_HALF_TEMP))

    accv = jnp.zeros((bi, cj), jnp.bfloat16)
    scnt = jnp.zeros((bi, 1), jnp.bfloat16)
    for bb in range(b):
        xab = xa_ref[bb]
        prior = jnp.zeros((bi, 1), jnp.bfloat16)
        for c in range(n // cj):
            sl = slice(c * cj, (c + 1) * cj)
            d2x = jax.lax.dot_general(xab, xbt_ref[bb][:, sl], dn,
                                      preferred_element_type=jnp.float32)
            wc = jnp.where(d2x < r2, noteye[:, sl], 0.0).astype(jnp.bfloat16)
            dxp = jnp.maximum(d2x.astype(jnp.bfloat16), tiny)
            tall = jnp.abs(dxp * jax.lax.rsqrt(dxp) - cnd[:, sl]) * sw[:, sl]
            incl = jax.lax.dot_general(wc, u, dn,
                                       preferred_element_type=jnp.float32
                                       ).astype(jnp.bfloat16)
            ind = wc * jnp.clip(twenty - (prior + incl), zero, one)
            accv = accv + tall * ind
            prior = jnp.minimum(prior + incl[:, cj - 1:cj], twenty)
        scnt = scnt + jnp.minimum(prior, jnp.bfloat16(_KEEP))
    out_ref[...] += (jnp.sum(accv.astype(jnp.float32), keepdims=True)
                     - 1e-10 * jnp.sum(scnt.astype(jnp.float32), keepdims=True))


def kernel(xyz, canno_xyz, radius):
    b, n, _ = xyz.shape
    bi, cj = 512, 256
    sq = jnp.sum(xyz * xyz, axis=-1, keepdims=True)
    ones = jnp.ones_like(sq)
    xa = jnp.concatenate([xyz, sq, ones], axis=-1)
    xbt = jnp.swapaxes(
        jnp.concatenate([-2.0 * xyz, ones, sq], axis=-1), 1, 2)
    xah = xa.astype(jnp.bfloat16)
    xal = (xa - xah.astype(jnp.float32)).astype(jnp.bfloat16)
    xbh = xbt.astype(jnp.bfloat16)
    xbl = (xbt - xbh.astype(jnp.float32)).astype(jnp.bfloat16)
    xa3 = jnp.concatenate([xah, xah, xal], axis=-1)
    xb3 = jnp.concatenate([xbh, xbl, xbh], axis=1)
    csq = jnp.sum(canno_xyz * canno_xyz, axis=-1, keepdims=True)
    cones = jnp.ones_like(csq)
    ca = jnp.concatenate([canno_xyz, csq, cones],
                         axis=-1).astype(jnp.bfloat16)
    cbt = jnp.concatenate([-2.0 * canno_xyz, cones, csq],
                          axis=-1).T.astype(jnp.bfloat16)
    r = jnp.asarray(radius, jnp.float32)
    r2 = jnp.reshape(r * r, (1, 1))
    u = jnp.triu(jnp.ones((cj, cj), jnp.bfloat16))
    out = pl.pallas_call(
        functools.partial(_loss_kernel, bi=bi, cj=cj, n=n, b=b),
        grid=(n // bi,),
        in_specs=[
            pl.BlockSpec((1, 1), lambda i: (0, 0)),
            pl.BlockSpec((b, bi, 15), lambda i: (0, i, 0)),
            pl.BlockSpec((b, 15, n), lambda i: (0, 0, 0)),
            pl.BlockSpec((bi, 5), lambda i: (i, 0)),
            pl.BlockSpec((5, n), lambda i: (0, 0)),
            pl.BlockSpec((cj, cj), lambda i: (0, 0)),
        ],
        out_specs=pl.BlockSpec((1, 1), lambda i: (0, 0)),
        out_shape=jax.ShapeDtypeStruct((1, 1), jnp.float32),
    )(r2, xa3, xb3, ca, cbt, u)
    denom = b * n * int(_KEEP)
    return (out[0, 0] / denom + 1e-10).astype(jnp.float32)
